# baseline (speedup 1.0000x reference)
import jax
import jax.numpy as jnp
from jax import lax
from jax.experimental import pallas as pl
from jax.experimental.pallas import tpu as pltpu
from jax.experimental.pallas import tpu_sc as plsc

N = 10000
E = 320000
D = 128
WS = 80
NC = 2
NS = 16
EW = E // NC // NS
EPT = E // NS
K = 80
NB = 5
RING = 5
LOOK = 2
NCH_DEG = EW // K
NCH_ACC = EPT // K
DEG_PAD = 10240
DEG_SLAB = DEG_PAD // NS
SLAB = N // NS

_MESH = dict(core_axis_name="c", subcore_axis_name="s", num_cores=NC,
             num_subcores=NS)
_SC_PARAMS = pltpu.CompilerParams(use_tc_tiling_on_sc=False)


def _make_deg():
    def body(ei_hbm, ones_hbm, zeros_hbm, out_hbm, idx_v, ones_v, acc_sh,
             sem):
        c = lax.axis_index("c")
        s = lax.axis_index("s")
        w = c * NS + s
        slab = s * DEG_SLAB
        pltpu.sync_copy(ones_hbm, ones_v)
        pltpu.sync_copy(ei_hbm.at[1, pl.ds(w * EW, EW)], idx_v)
        pltpu.sync_copy(zeros_hbm, acc_sh.at[pl.ds(slab, DEG_SLAB)])
        plsc.subcore_barrier()

        def group(go, carry):
            g0 = go * NB
            descs = [pltpu.async_copy(
                ones_v, acc_sh.at[idx_v.at[pl.ds((g0 + b) * K, K)]], sem,
                add=True) for b in range(NB)]
            for d in descs:
                d.wait()
            return carry

        lax.fori_loop(0, NCH_DEG // NB, group, 0)
        plsc.subcore_barrier()
        pltpu.sync_copy(acc_sh.at[pl.ds(slab, DEG_SLAB)],
                        out_hbm.at[c, pl.ds(slab, DEG_SLAB)])

    return pl.kernel(
        body,
        out_type=jax.ShapeDtypeStruct((NC, DEG_PAD, 16), jnp.float32),
        mesh=plsc.VectorSubcoreMesh(**_MESH),
        compiler_params=_SC_PARAMS,
        scratch_types=[
            pltpu.VMEM((EW,), jnp.int32),
            pltpu.VMEM((K, 16), jnp.float32),
            pltpu.VMEM_SHARED((DEG_PAD, 16), jnp.float32),
            pltpu.SemaphoreType.DMA,
        ],
    )


def _make_accum():
    def body(y_hbm, col2_hbm, ei_hbm, zeros_hbm, out_hbm,
             idxc, idxr, bufs, acc_sh, gsem, ssem):
        c = lax.axis_index("c")
        s = lax.axis_index("s")
        slab = s * SLAB
        pltpu.sync_copy(col2_hbm.at[pl.ds(c * E + s * EPT, EPT)], idxc)
        pltpu.sync_copy(ei_hbm.at[0, pl.ds(s * EPT, EPT)], idxr)
        pltpu.sync_copy(zeros_hbm, acc_sh.at[pl.ds(slab, SLAB)])
        plsc.subcore_barrier()

        def gather(g, b):
            return pltpu.async_copy(
                y_hbm.at[idxc.at[pl.ds(g * K, K)]], bufs.at[b], gsem)

        for g in range(LOOK):
            gather(g, g % RING)

        def step(i, carry):
            for u in range(RING):
                g = i * RING + u
                pltpu.make_async_copy(y_hbm.at[idxc.at[pl.ds(g * K, K)]],
                                      bufs.at[u], gsem).wait()
                pltpu.async_copy(bufs.at[u],
                                 acc_sh.at[idxr.at[pl.ds(g * K, K)]], ssem,
                                 add=True)

                @pl.when(g >= RING - LOOK)
                def _():
                    pltpu.make_async_copy(
                        bufs.at[(u + LOOK) % RING],
                        acc_sh.at[idxr.at[pl.ds(g * K, K)]], ssem).wait()

                @pl.when(g + LOOK < NCH_ACC)
                def _():
                    gather(g + LOOK, (u + LOOK) % RING)
            return carry

        lax.fori_loop(0, NCH_ACC // RING, step, 0)
        for _g in range(RING - LOOK):
            pltpu.make_async_copy(bufs.at[0],
                                  acc_sh.at[idxr.at[pl.ds(0, K)]],
                                  ssem).wait()
        plsc.subcore_barrier()
        pltpu.sync_copy(acc_sh.at[pl.ds(slab, SLAB)],
                        out_hbm.at[c, pl.ds(slab, SLAB), pl.ds(0, WS)])

    return pl.kernel(
        body,
        out_type=jax.ShapeDtypeStruct((NC, N, D), jnp.float32),
        mesh=plsc.VectorSubcoreMesh(**_MESH),
        compiler_params=_SC_PARAMS,
        scratch_types=[
            pltpu.VMEM((EPT,), jnp.int32),
            pltpu.VMEM((EPT,), jnp.int32),
            pltpu.VMEM((RING, K, WS), jnp.float32),
            pltpu.VMEM_SHARED((N, WS), jnp.float32),
            pltpu.SemaphoreType.DMA,
            pltpu.SemaphoreType.DMA,
        ],
    )


_RB = 1000


def _build_y_body(parts_ref, x_ref, y_ref):
    j = pl.program_id(0)
    deg = parts_ref[0, :, 0:1] + parts_ref[1, :, 0:1]
    dis = jnp.where(deg > 0, lax.rsqrt(jnp.maximum(deg, 1.0)), 0.0)
    xv = x_ref[...]
    v0 = xv[:, :WS] * dis
    lane = lax.broadcasted_iota(jnp.int32, (_RB, WS - (D - WS)), 1)
    v1 = jnp.concatenate(
        [xv[:, WS:D] * dis, jnp.where(lane == 0, dis, 0.0)], axis=1)
    y_ref[...] = jnp.where(j >= N // _RB, v1, v0)


def _build_y(parts, x):
    nb = N // _RB
    return pl.pallas_call(
        _build_y_body,
        grid=(2 * nb,),
        in_specs=[
            pl.BlockSpec((NC, _RB, 16), lambda j: (0, j % nb, 0)),
            pl.BlockSpec((_RB, D), lambda j: (j % nb, 0)),
        ],
        out_specs=pl.BlockSpec((_RB, WS), lambda j: (j, 0)),
        out_shape=jax.ShapeDtypeStruct((2 * N, WS), jnp.float32),
    )(parts, x)


def _finish_body(ts_ref, y_ref, x_ref, w1_ref, b1_ref, w2_ref, b2_ref,
                 out_ref):
    T0 = ts_ref[0, :, :WS]
    T1 = ts_ref[1, :, :WS]
    dis = y_ref[:, (D - WS):(D - WS) + 1]
    T = jnp.concatenate([T0, T1[:, :D - WS]], axis=1)
    S = T * dis
    cc = T1[:, (D - WS):(D - WS) + 1] * dis
    dn = (((1,), (1,)), ((), ()))
    h = lax.dot_general(S, w1_ref[...], dn,
                        precision=lax.Precision.HIGHEST,
                        preferred_element_type=jnp.float32)
    h = h + lax.dot_general(x_ref[...] * S, w2_ref[...], dn,
                            precision=lax.Precision.HIGHEST,
                            preferred_element_type=jnp.float32)
    h = h + cc * (b1_ref[...] + b2_ref[...])[None, :]
    out_ref[...] = jnp.where(h >= 0, h, 0.2 * h)


def _finish(tsplit, y, x, W1_w, W1_b, W2_w, W2_b):
    nb = N // _RB
    return pl.pallas_call(
        _finish_body,
        grid=(nb,),
        in_specs=[
            pl.BlockSpec((NC, _RB, D), lambda i: (0, i, 0)),
            pl.BlockSpec((_RB, WS), lambda i: (i + nb, 0)),
            pl.BlockSpec((_RB, D), lambda i: (i, 0)),
            pl.BlockSpec((D, D), lambda i: (0, 0)),
            pl.BlockSpec((D,), lambda i: (0,)),
            pl.BlockSpec((D, D), lambda i: (0, 0)),
            pl.BlockSpec((D,), lambda i: (0,)),
        ],
        out_specs=pl.BlockSpec((_RB, D), lambda i: (i, 0)),
        out_shape=jax.ShapeDtypeStruct((N, D), jnp.float32),
    )(tsplit, y, x, W1_w, W1_b, W2_w, W2_b)


def kernel(x, edge_index, W1_w, W1_b, W2_w, W2_b):
    ei = edge_index.astype(jnp.int32)
    col = ei[1]
    col2 = jnp.concatenate([col, col + N])
    ones16 = jnp.ones((K, 16), jnp.float32)
    zeros16 = jnp.zeros((DEG_SLAB, 16), jnp.float32)
    zerosWS = jnp.zeros((SLAB, WS), jnp.float32)

    deg_parts = _make_deg()(ei, ones16, zeros16)
    y = _build_y(deg_parts, x)
    tsplit = _make_accum()(y, col2, ei, zerosWS)
    return _finish(tsplit, y, x, W1_w, W1_b, W2_w, W2_b)
